# Initial kernel scaffold; baseline (speedup 1.0000x reference)
import jax
import jax.numpy as jnp
from jax.experimental import pallas as pl


def kernel(x, W1, b1, W2, b2, gamma, beta, edge_index):
    raise NotImplementedError("write your pallas kernel here")


# speedup vs baseline: 4.9051x; 4.9051x over previous
import functools

import jax
import jax.numpy as jnp
from jax import lax
from jax.experimental import pallas as pl
from jax.experimental.pallas import tpu as pltpu
from jax.experimental.pallas import tpu_sc as plsc

BN_EPS = 1e-5


def _make_sc_aggregate(N, D, E):
    info = plsc.get_sparse_core_info()
    NC, NS = info.num_cores, info.num_subcores
    NW = NC * NS
    assert E % NW == 0
    epw = E // NW
    chunk = 80
    assert epw % chunk == 0
    nchunk = epw // chunk
    rps = (N // NS) // 8 * 8
    rem = N - NS * rps
    assert rem % 8 == 0

    mesh = plsc.VectorSubcoreMesh(core_axis_name="c", subcore_axis_name="s")

    @functools.partial(
        pl.kernel,
        mesh=mesh,
        out_type=jax.ShapeDtypeStruct((NC, N, D), jnp.float32),
        scratch_types=[
            pltpu.VMEM((chunk,), jnp.int32),
            pltpu.VMEM((chunk,), jnp.int32),
            pltpu.VMEM((chunk, D), jnp.float32),
            pltpu.VMEM_SHARED((N, D), jnp.float32),
        ],
    )
    def agg_kernel(x_hbm, src_hbm, dst_hbm, zero_hbm, out_hbm,
                   src_idx, dst_idx, rows, acc):
        c = lax.axis_index("c")
        s = lax.axis_index("s")
        wid = s * NC + c

        pltpu.sync_copy(zero_hbm.at[pl.ds(s * rps, rps)],
                        acc.at[pl.ds(s * rps, rps)])

        @pl.when(s == NS - 1)
        def _():
            pltpu.sync_copy(zero_hbm.at[pl.ds(NS * rps, rem)],
                            acc.at[pl.ds(NS * rps, rem)])

        plsc.subcore_barrier()

        base = wid * epw

        def chunk_body(i, carry):
            off = base + i * chunk
            pltpu.sync_copy(src_hbm.at[pl.ds(off, chunk)], src_idx)
            pltpu.sync_copy(dst_hbm.at[pl.ds(off, chunk)], dst_idx)
            pltpu.sync_copy(x_hbm.at[src_idx], rows)
            pltpu.sync_copy(rows, acc.at[dst_idx], add=True)
            return carry

        lax.fori_loop(0, nchunk, chunk_body, 0)
        plsc.subcore_barrier()

        pltpu.sync_copy(acc.at[pl.ds(s * rps, rps)],
                        out_hbm.at[c, pl.ds(s * rps, rps)])

        @pl.when(s == NS - 1)
        def _():
            pltpu.sync_copy(acc.at[pl.ds(NS * rps, rem)],
                            out_hbm.at[c, pl.ds(NS * rps, rem)])

    return agg_kernel


def _mlp_bn_body(x_ref, agg_ref, w1_ref, b1_ref, w2_ref, b2_ref,
                 g_ref, beta_ref, o_ref):
    h = x_ref[...] + agg_ref[0] + agg_ref[1]
    h = jnp.dot(h, w1_ref[...], preferred_element_type=jnp.float32)
    h = jnp.maximum(h + b1_ref[...], 0.0)
    h = jnp.dot(h, w2_ref[...], preferred_element_type=jnp.float32)
    h = jnp.maximum(h + b2_ref[...], 0.0)
    mean = jnp.mean(h, axis=0, keepdims=True)
    d = h - mean
    var = jnp.mean(d * d, axis=0, keepdims=True)
    o_ref[...] = g_ref[...] * d * lax.rsqrt(var + BN_EPS) + beta_ref[...]


def kernel(x, W1, b1, W2, b2, gamma, beta, edge_index):
    N, D = x.shape
    H = W1.shape[1]
    E = edge_index.shape[1]

    src = edge_index[0]
    dst = edge_index[1]
    zeros = jnp.zeros((N, D), dtype=jnp.float32)

    agg2 = _make_sc_aggregate(N, D, E)(x, src, dst, zeros)

    out = pl.pallas_call(
        _mlp_bn_body,
        out_shape=jax.ShapeDtypeStruct((N, H), jnp.float32),
    )(x, agg2, W1, b1.reshape(1, H), W2, b2.reshape(1, H),
      gamma.reshape(1, H), beta.reshape(1, H))
    return out


# speedup vs baseline: 9.7826x; 1.9944x over previous
import functools

import jax
import jax.numpy as jnp
from jax import lax
from jax.experimental import pallas as pl
from jax.experimental.pallas import tpu as pltpu
from jax.experimental.pallas import tpu_sc as plsc

BN_EPS = 1e-5


def _make_sc_aggregate(N, D, E):
    info = plsc.get_sparse_core_info()
    NC, NS = info.num_cores, info.num_subcores
    NW = NC * NS
    assert E % NW == 0
    epw = E // NW
    chunk = 80
    assert epw % chunk == 0
    nchunk = epw // chunk
    cpb = 25
    assert nchunk % cpb == 0 and cpb % 2 == 1
    nblk = nchunk // cpb
    rps = (N // NS) // 8 * 8
    rem = N - NS * rps
    assert rem % 8 == 0

    mesh = plsc.VectorSubcoreMesh(core_axis_name="c", subcore_axis_name="s")

    @functools.partial(
        pl.kernel,
        mesh=mesh,
        out_type=jax.ShapeDtypeStruct((NC, N, D), jnp.float32),
        scratch_types=[
            pltpu.VMEM((cpb, chunk), jnp.int32),
            pltpu.VMEM((cpb, chunk), jnp.int32),
            pltpu.VMEM((chunk, D), jnp.float32),
            pltpu.VMEM((chunk, D), jnp.float32),
            pltpu.VMEM_SHARED((N, D), jnp.float32),
            pltpu.SemaphoreType.DMA,
            pltpu.SemaphoreType.DMA,
        ],
    )
    def agg_kernel(x_hbm, src_hbm, dst_hbm, zero_hbm, out_hbm,
                   src_idx, dst_idx, rows0, rows1, acc, sem0, sem1):
        c = lax.axis_index("c")
        s = lax.axis_index("s")
        wid = s * NC + c

        @pl.when(c == 0)
        def _():
            pltpu.sync_copy(x_hbm.at[pl.ds(s * rps, rps)],
                            acc.at[pl.ds(s * rps, rps)])

            @pl.when(s == NS - 1)
            def _():
                pltpu.sync_copy(x_hbm.at[pl.ds(NS * rps, rem)],
                                acc.at[pl.ds(NS * rps, rem)])

        @pl.when(c != 0)
        def _():
            pltpu.sync_copy(zero_hbm.at[pl.ds(s * rps, rps)],
                            acc.at[pl.ds(s * rps, rps)])

            @pl.when(s == NS - 1)
            def _():
                pltpu.sync_copy(zero_hbm.at[pl.ds(NS * rps, rem)],
                                acc.at[pl.ds(NS * rps, rem)])

        plsc.subcore_barrier()

        def blk_body(blk, carry0):
            pltpu.sync_copy(src_hbm.at[wid, blk], src_idx)
            pltpu.sync_copy(dst_hbm.at[wid, blk], dst_idx)
            pltpu.async_copy(x_hbm.at[src_idx.at[0]], rows0, sem0)

            def pair_body(i, carry):
                a = 2 * i
                pltpu.async_copy(x_hbm.at[src_idx.at[a + 1]], rows1, sem1)
                pltpu.make_async_copy(x_hbm.at[src_idx.at[a]], rows0,
                                      sem0).wait()
                pltpu.sync_copy(rows0, acc.at[dst_idx.at[a]], add=True)
                pltpu.async_copy(x_hbm.at[src_idx.at[a + 2]], rows0, sem0)
                pltpu.make_async_copy(x_hbm.at[src_idx.at[a + 1]], rows1,
                                      sem1).wait()
                pltpu.sync_copy(rows1, acc.at[dst_idx.at[a + 1]], add=True)
                return carry

            lax.fori_loop(0, (cpb - 1) // 2, pair_body, 0)
            pltpu.make_async_copy(x_hbm.at[src_idx.at[cpb - 1]], rows0,
                                  sem0).wait()
            pltpu.sync_copy(rows0, acc.at[dst_idx.at[cpb - 1]], add=True)
            return carry0

        lax.fori_loop(0, nblk, blk_body, 0)

        plsc.subcore_barrier()

        pltpu.sync_copy(acc.at[pl.ds(s * rps, rps)],
                        out_hbm.at[c, pl.ds(s * rps, rps)])

        @pl.when(s == NS - 1)
        def _():
            pltpu.sync_copy(acc.at[pl.ds(NS * rps, rem)],
                            out_hbm.at[c, pl.ds(NS * rps, rem)])

    return agg_kernel


def _mlp_bn_body(agg_ref, w1_ref, b1_ref, w2_ref, b2_ref,
                 g_ref, beta_ref, o_ref):
    h = agg_ref[0] + agg_ref[1]
    h = jnp.dot(h, w1_ref[...], preferred_element_type=jnp.float32)
    h = jnp.maximum(h + b1_ref[...], 0.0)
    h = jnp.dot(h, w2_ref[...], preferred_element_type=jnp.float32)
    h = jnp.maximum(h + b2_ref[...], 0.0)
    mean = jnp.mean(h, axis=0, keepdims=True)
    d = h - mean
    var = jnp.mean(d * d, axis=0, keepdims=True)
    o_ref[...] = g_ref[...] * d * lax.rsqrt(var + BN_EPS) + beta_ref[...]


def kernel(x, W1, b1, W2, b2, gamma, beta, edge_index):
    N, D = x.shape
    H = W1.shape[1]
    E = edge_index.shape[1]

    info = plsc.get_sparse_core_info()
    NW = info.num_cores * info.num_subcores
    epw = E // NW
    chunk = 80
    cpb = 25
    nblk = epw // (chunk * cpb)

    src = edge_index[0].reshape(NW, nblk, cpb, chunk)
    dst = edge_index[1].reshape(NW, nblk, cpb, chunk)
    zeros = jnp.zeros((N, D), dtype=jnp.float32)

    agg2 = _make_sc_aggregate(N, D, E)(x, src, dst, zeros)

    out = pl.pallas_call(
        _mlp_bn_body,
        out_shape=jax.ShapeDtypeStruct((N, H), jnp.float32),
    )(agg2, W1, b1.reshape(1, H), W2, b2.reshape(1, H),
      gamma.reshape(1, H), beta.reshape(1, H))
    return out
